# baseline (speedup 1.0000x reference)
import functools

import jax
import jax.numpy as jnp
from jax.experimental import pallas as pl
from jax.experimental.pallas import tpu as pltpu
from jax.experimental.pallas import tpu_sc as plsc

_M = 16384
_K = 256
_N = 8192
_BM = 256
_GW = 128


def _dist_argmin_body(z_ref, w_ref, idx_ref):
    zb = z_ref[...]
    wb = w_ref[...]
    mm = jax.lax.dot_general(
        zb, wb, (((1,), (1,)), ((), ())),
        preferred_element_type=jnp.float32)
    zn = jnp.sum(zb * zb, axis=1, keepdims=True)
    ones = jnp.ones((1, _K), jnp.float32)
    wn = jax.lax.dot_general(
        ones, wb * wb, (((1,), (1,)), ((), ())),
        precision=jax.lax.Precision.HIGHEST,
        preferred_element_type=jnp.float32)
    d = zn - 2.0 * mm + wn
    dmin = jnp.min(d, axis=1, keepdims=True)
    iota = jax.lax.broadcasted_iota(jnp.int32, (_BM, _N), 1)
    idx_ref[0, 0, :] = jnp.min(jnp.where(d == dmin, iota, _N), axis=1)


def _encode(z, weight):
    idx3 = pl.pallas_call(
        _dist_argmin_body,
        grid=(_M // _BM,),
        in_specs=[
            pl.BlockSpec((_BM, _K), lambda i: (i, 0)),
            pl.BlockSpec((_N, _K), lambda i: (0, 0)),
        ],
        out_specs=pl.BlockSpec((1, 1, _BM), lambda i: (i, 0, 0)),
        out_shape=jax.ShapeDtypeStruct((_M // _BM, 1, _BM), jnp.int32),
    )(z, weight)
    return idx3.reshape(_M)


def _gather_rows(weight, indices):
    idx2 = indices.reshape(1, _M)
    mesh = plsc.VectorSubcoreMesh(
        core_axis_name="core", subcore_axis_name="subcore")

    @functools.partial(
        pl.kernel,
        out_type=jax.ShapeDtypeStruct((_M, _K), jnp.float32),
        mesh=mesh)
    def _kern(w_hbm, i_hbm, o_hbm):
        def body(i_vmem, o_vmem):
            pltpu.sync_copy(w_hbm.at[i_vmem.at[0]], o_vmem)

        pltpu.emit_pipeline(
            body,
            grid=(_M // _GW,),
            in_specs=[pl.BlockSpec((1, _GW), index_map=lambda i: (0, i))],
            out_specs=[pl.BlockSpec((_GW, _K), index_map=lambda i: (i, 0))],
            core_axis_name=("core", "subcore"),
            dimension_semantics=(pltpu.PARALLEL,),
        )(i_hbm, o_hbm)

    return _kern(weight, idx2)


def kernel(z, weight):
    flat = z.reshape(-1, _K)
    encoding_indices = _encode(flat, weight)
    quantized = _gather_rows(weight, encoding_indices)
    return quantized, encoding_indices


# speedup vs baseline: 2.6062x; 2.6062x over previous
import functools

import jax
import jax.numpy as jnp
from jax.experimental import pallas as pl
from jax.experimental.pallas import tpu as pltpu
from jax.experimental.pallas import tpu_sc as plsc

_M = 16384
_K = 256
_N = 8192
_BM = 256
_GW = 128


def _dist_argmin_body(z_ref, zm2_ref, w_ref, w16_ref, idx_ref, wn_ref):
    @pl.when(pl.program_id(0) == 0)
    def _():
        ones = jnp.ones((1, _K), jnp.float32)
        wn_ref[...] = jax.lax.dot_general(
            ones, w_ref[...] * w_ref[...], (((1,), (1,)), ((), ())),
            precision=jax.lax.Precision.HIGHEST,
            preferred_element_type=jnp.float32)

    zb = z_ref[...]
    mm2 = jax.lax.dot_general(
        zm2_ref[...], w16_ref[...], (((1,), (1,)), ((), ())),
        preferred_element_type=jnp.float32)
    zn = jnp.sum(zb * zb, axis=1, keepdims=True)
    d = (zn + mm2) + wn_ref[...]
    dmin = jnp.min(d, axis=1, keepdims=True)
    iota = jax.lax.broadcasted_iota(jnp.int32, (_BM, _N), 1)
    idx_ref[0, 0, :] = jnp.min(jnp.where(d == dmin, iota, _N), axis=1)


def _encode(z, weight):
    zm2 = (-2.0 * z).astype(jnp.bfloat16)
    w16 = weight.astype(jnp.bfloat16)
    idx3 = pl.pallas_call(
        _dist_argmin_body,
        grid=(_M // _BM,),
        in_specs=[
            pl.BlockSpec((_BM, _K), lambda i: (i, 0)),
            pl.BlockSpec((_BM, _K), lambda i: (i, 0)),
            pl.BlockSpec((_N, _K), lambda i: (0, 0)),
            pl.BlockSpec((_N, _K), lambda i: (0, 0)),
        ],
        out_specs=pl.BlockSpec((1, 1, _BM), lambda i: (i, 0, 0)),
        out_shape=jax.ShapeDtypeStruct((_M // _BM, 1, _BM), jnp.int32),
        scratch_shapes=[pltpu.VMEM((1, _N), jnp.float32)],
    )(z, zm2, weight, w16)
    return idx3.reshape(_M)


def _gather_rows(weight, indices):
    idx2 = indices.reshape(1, _M)
    mesh = plsc.VectorSubcoreMesh(
        core_axis_name="core", subcore_axis_name="subcore")

    @functools.partial(
        pl.kernel,
        out_type=jax.ShapeDtypeStruct((_M, _K), jnp.float32),
        mesh=mesh)
    def _kern(w_hbm, i_hbm, o_hbm):
        def body(i_vmem, o_vmem):
            pltpu.sync_copy(w_hbm.at[i_vmem.at[0]], o_vmem)

        pltpu.emit_pipeline(
            body,
            grid=(_M // _GW,),
            in_specs=[pl.BlockSpec((1, _GW), index_map=lambda i: (0, i))],
            out_specs=[pl.BlockSpec((_GW, _K), index_map=lambda i: (i, 0))],
            core_axis_name=("core", "subcore"),
            dimension_semantics=(pltpu.PARALLEL,),
        )(i_hbm, o_hbm)

    return _kern(weight, idx2)


def kernel(z, weight):
    flat = z.reshape(-1, _K)
    encoding_indices = _encode(flat, weight)
    quantized = _gather_rows(weight, encoding_indices)
    return quantized, encoding_indices


# speedup vs baseline: 3.2756x; 1.2568x over previous
import functools

import jax
import jax.numpy as jnp
from jax.experimental import pallas as pl
from jax.experimental.pallas import tpu as pltpu
from jax.experimental.pallas import tpu_sc as plsc

_M = 16384
_K = 256
_N = 8192
_BM = 256
_GW = 128


def _dist_argmin_body(zt_ref, zm2_ref, w_ref, w16_ref, idx_ref, wn_ref):
    @pl.when(pl.program_id(0) == 0)
    def _():
        wb = w_ref[...]
        wn_ref[...] = jnp.sum(wb * wb, axis=1, keepdims=True)

    mm2 = jax.lax.dot_general(
        w16_ref[...], zm2_ref[...], (((1,), (0,)), ((), ())),
        preferred_element_type=jnp.float32)
    zt = zt_ref[...]
    zn = jnp.sum(zt * zt, axis=0, keepdims=True)

    _C = 64
    bestv = None
    for c in range(_N // _C):
        d = (zn + mm2[c * _C:(c + 1) * _C, :]) + wn_ref[c * _C:(c + 1) * _C, :]
        if bestv is None:
            bestv = d
            bestr = jnp.zeros((_C, _BM), jnp.int32)
        else:
            m = d < bestv
            bestv = jnp.where(m, d, bestv)
            bestr = jnp.where(m, c, bestr)

    dmin = jnp.min(bestv, axis=0, keepdims=True)
    srow = jax.lax.broadcasted_iota(jnp.int32, (_C, _BM), 0)
    n_mat = bestr * _C + srow
    idx_ref[0, 0, :] = jnp.min(
        jnp.where(bestv == dmin, n_mat, _N), axis=0)


def _encode(z, weight):
    zt = z.T
    zm2 = (-2.0 * z).astype(jnp.bfloat16).T
    w16 = weight.astype(jnp.bfloat16)
    idx3 = pl.pallas_call(
        _dist_argmin_body,
        grid=(_M // _BM,),
        in_specs=[
            pl.BlockSpec((_K, _BM), lambda i: (0, i)),
            pl.BlockSpec((_K, _BM), lambda i: (0, i)),
            pl.BlockSpec((_N, _K), lambda i: (0, 0)),
            pl.BlockSpec((_N, _K), lambda i: (0, 0)),
        ],
        out_specs=pl.BlockSpec((1, 1, _BM), lambda i: (i, 0, 0)),
        out_shape=jax.ShapeDtypeStruct((_M // _BM, 1, _BM), jnp.int32),
        scratch_shapes=[pltpu.VMEM((_N, 1), jnp.float32)],
    )(zt, zm2, weight, w16)
    return idx3.reshape(_M)


def _gather_rows(weight, indices):
    idx2 = indices.reshape(1, _M)
    mesh = plsc.VectorSubcoreMesh(
        core_axis_name="core", subcore_axis_name="subcore")

    @functools.partial(
        pl.kernel,
        out_type=jax.ShapeDtypeStruct((_M, _K), jnp.float32),
        mesh=mesh)
    def _kern(w_hbm, i_hbm, o_hbm):
        def body(i_vmem, o_vmem):
            pltpu.sync_copy(w_hbm.at[i_vmem.at[0]], o_vmem)

        pltpu.emit_pipeline(
            body,
            grid=(_M // _GW,),
            in_specs=[pl.BlockSpec((1, _GW), index_map=lambda i: (0, i))],
            out_specs=[pl.BlockSpec((_GW, _K), index_map=lambda i: (i, 0))],
            core_axis_name=("core", "subcore"),
            dimension_semantics=(pltpu.PARALLEL,),
        )(i_hbm, o_hbm)

    return _kern(weight, idx2)


def kernel(z, weight):
    flat = z.reshape(-1, _K)
    encoding_indices = _encode(flat, weight)
    quantized = _gather_rows(weight, encoding_indices)
    return quantized, encoding_indices


# speedup vs baseline: 3.3473x; 1.0219x over previous
import functools

import jax
import jax.numpy as jnp
from jax.experimental import pallas as pl
from jax.experimental.pallas import tpu as pltpu
from jax.experimental.pallas import tpu_sc as plsc

_M = 16384
_K = 256
_N = 8192
_BM = 256
_GW = 128


def _dist_argmin_body(z_ref, zm2_ref, w_ref, w16_ref, idx_ref, wn_ref):
    @pl.when(pl.program_id(0) == 0)
    def _():
        wb = w_ref[...]
        wn_ref[...] = jnp.sum(wb * wb, axis=1, keepdims=True)

    zb = z_ref[...]
    mm2 = jax.lax.dot_general(
        w16_ref[...], zm2_ref[...], (((1,), (0,)), ((), ())),
        preferred_element_type=jnp.float32)
    zn = jnp.transpose(jnp.sum(zb * zb, axis=1, keepdims=True))

    _C = 64
    bestv = None
    for c in range(_N // _C):
        d = (zn + mm2[c * _C:(c + 1) * _C, :]) + wn_ref[c * _C:(c + 1) * _C, :]
        if bestv is None:
            bestv = d
            bestr = jnp.zeros((_C, _BM), jnp.int32)
        else:
            m = d < bestv
            bestv = jnp.where(m, d, bestv)
            bestr = jnp.where(m, c, bestr)

    dmin = jnp.min(bestv, axis=0, keepdims=True)
    srow = jax.lax.broadcasted_iota(jnp.int32, (_C, _BM), 0)
    n_mat = bestr * _C + srow
    idx_ref[0, 0, :] = jnp.min(
        jnp.where(bestv == dmin, n_mat, _N), axis=0)


def _encode(z, weight):
    zm2 = (-2.0 * z).astype(jnp.bfloat16).T
    w16 = weight.astype(jnp.bfloat16)
    idx3 = pl.pallas_call(
        _dist_argmin_body,
        grid=(_M // _BM,),
        in_specs=[
            pl.BlockSpec((_BM, _K), lambda i: (i, 0)),
            pl.BlockSpec((_K, _BM), lambda i: (0, i)),
            pl.BlockSpec((_N, _K), lambda i: (0, 0)),
            pl.BlockSpec((_N, _K), lambda i: (0, 0)),
        ],
        out_specs=pl.BlockSpec((1, 1, _BM), lambda i: (i, 0, 0)),
        out_shape=jax.ShapeDtypeStruct((_M // _BM, 1, _BM), jnp.int32),
        scratch_shapes=[pltpu.VMEM((_N, 1), jnp.float32)],
    )(z, zm2, weight, w16)
    return idx3.reshape(_M)


def _gather_rows(weight, indices):
    idx2 = indices.reshape(1, _M)
    mesh = plsc.VectorSubcoreMesh(
        core_axis_name="core", subcore_axis_name="subcore")

    @functools.partial(
        pl.kernel,
        out_type=jax.ShapeDtypeStruct((_M, _K), jnp.float32),
        mesh=mesh)
    def _kern(w_hbm, i_hbm, o_hbm):
        def body(i_vmem, o_vmem):
            pltpu.sync_copy(w_hbm.at[i_vmem.at[0]], o_vmem)

        pltpu.emit_pipeline(
            body,
            grid=(_M // _GW,),
            in_specs=[pl.BlockSpec((1, _GW), index_map=lambda i: (0, i))],
            out_specs=[pl.BlockSpec((_GW, _K), index_map=lambda i: (i, 0))],
            core_axis_name=("core", "subcore"),
            dimension_semantics=(pltpu.PARALLEL,),
        )(i_hbm, o_hbm)

    return _kern(weight, idx2)


def kernel(z, weight):
    flat = z.reshape(-1, _K)
    encoding_indices = _encode(flat, weight)
    quantized = _gather_rows(weight, encoding_indices)
    return quantized, encoding_indices


# speedup vs baseline: 3.9452x; 1.1786x over previous
import functools

import jax
import jax.numpy as jnp
from jax.experimental import pallas as pl
from jax.experimental.pallas import tpu as pltpu
from jax.experimental.pallas import tpu_sc as plsc

_M = 16384
_K = 256
_N = 8192
_BM = 256
_GW = 128


def _dist_argmin_body(z_ref, w_ref, idx_ref, wn_ref, w16_ref):
    @pl.when(pl.program_id(0) == 0)
    def _():
        wb = w_ref[...]
        wn_ref[...] = jnp.sum(wb * wb, axis=1, keepdims=True)
        w16_ref[...] = wb.astype(jnp.bfloat16)

    zb = z_ref[...]
    zm2 = (-2.0 * zb).astype(jnp.bfloat16)
    mm2 = jax.lax.dot_general(
        w16_ref[...], zm2, (((1,), (1,)), ((), ())),
        preferred_element_type=jnp.float32)
    zn = jnp.transpose(jnp.sum(zb * zb, axis=1, keepdims=True))

    _C = 64
    bestv = None
    for c in range(_N // _C):
        d = (zn + mm2[c * _C:(c + 1) * _C, :]) + wn_ref[c * _C:(c + 1) * _C, :]
        if bestv is None:
            bestv = d
            bestr = jnp.zeros((_C, _BM), jnp.int32)
        else:
            m = d < bestv
            bestv = jnp.where(m, d, bestv)
            bestr = jnp.where(m, c, bestr)

    dmin = jnp.min(bestv, axis=0, keepdims=True)
    srow = jax.lax.broadcasted_iota(jnp.int32, (_C, _BM), 0)
    n_mat = bestr * _C + srow
    idx_ref[0, 0, :] = jnp.min(
        jnp.where(bestv == dmin, n_mat, _N), axis=0)


def _encode(z, weight):
    idx3 = pl.pallas_call(
        _dist_argmin_body,
        grid=(_M // _BM,),
        in_specs=[
            pl.BlockSpec((_BM, _K), lambda i: (i, 0)),
            pl.BlockSpec((_N, _K), lambda i: (0, 0)),
        ],
        out_specs=pl.BlockSpec((1, 1, _BM), lambda i: (i, 0, 0)),
        out_shape=jax.ShapeDtypeStruct((_M // _BM, 1, _BM), jnp.int32),
        scratch_shapes=[
            pltpu.VMEM((_N, 1), jnp.float32),
            pltpu.VMEM((_N, _K), jnp.bfloat16),
        ],
    )(z, weight)
    return idx3.reshape(_M)


def _gather_rows(weight, indices):
    idx2 = indices.reshape(1, _M)
    mesh = plsc.VectorSubcoreMesh(
        core_axis_name="core", subcore_axis_name="subcore")

    @functools.partial(
        pl.kernel,
        out_type=jax.ShapeDtypeStruct((_M, _K), jnp.float32),
        mesh=mesh)
    def _kern(w_hbm, i_hbm, o_hbm):
        def body(i_vmem, o_vmem):
            pltpu.sync_copy(w_hbm.at[i_vmem.at[0]], o_vmem)

        pltpu.emit_pipeline(
            body,
            grid=(_M // _GW,),
            in_specs=[pl.BlockSpec((1, _GW), index_map=lambda i: (0, i))],
            out_specs=[pl.BlockSpec((_GW, _K), index_map=lambda i: (i, 0))],
            core_axis_name=("core", "subcore"),
            dimension_semantics=(pltpu.PARALLEL,),
        )(i_hbm, o_hbm)

    return _kern(weight, idx2)


def kernel(z, weight):
    flat = z.reshape(-1, _K)
    encoding_indices = _encode(flat, weight)
    quantized = _gather_rows(weight, encoding_indices)
    return quantized, encoding_indices
